# baseline (speedup 1.0000x reference)
import dataclasses
import functools

import jax
import jax.numpy as jnp
from jax import lax
from jax.experimental import pallas as pl
from jax.experimental.pallas import tpu as pltpu
from jax.experimental.pallas import tpu_sc as plsc

N_BINS = 20
STRENGTH = 0.7

BS = 8
TOTAL = BS * 128 * 128 * 128
NC, NS, L = 2, 16, 16
NW = NC * NS
BLK = 16384
NROWS = TOTAL // BLK
ROWS_PER_BATCH = NROWS // BS


def _table_body(ry_ref, rysh_ref, slope_ref, icept_ref):
    col = lax.broadcasted_iota(jnp.int32, (BS, 128), 1).astype(jnp.float32)
    step = jnp.float32(1.0 / (N_BINS - 1))
    lin0 = col * step
    lin1 = (col + 1.0) * step
    w = jnp.float32(STRENGTH)
    y0 = ry_ref[...] * w + lin0 * (1.0 - w)
    y1 = rysh_ref[...] * w + lin1 * (1.0 - w)
    valid = col < float(N_BINS)
    big = jnp.float32(1e30)
    ymin = jnp.min(jnp.where(valid, y0, big), axis=1, keepdims=True)
    ymax = jnp.max(jnp.where(valid, y0, -big), axis=1, keepdims=True)
    d = ymax - ymin + jnp.float32(1e-5)
    y0n = (y0 - ymin) / d
    y1n = (y1 - ymin) / d
    slope = (y1n - y0n) / (lin1 - lin0)
    slope_ref[...] = slope
    icept_ref[...] = y0n - slope * lin0


def _make_tables(ran_y):
    ry = jnp.zeros((BS, 128), jnp.float32).at[:, :N_BINS].set(ran_y)
    rysh = jnp.zeros((BS, 128), jnp.float32).at[:, : N_BINS - 1].set(ran_y[:, 1:])
    return pl.pallas_call(
        _table_body,
        out_shape=(
            jax.ShapeDtypeStruct((BS, 128), jnp.float32),
            jax.ShapeDtypeStruct((BS, 128), jnp.float32),
        ),
    )(ry, rysh)


@functools.cache
def _build_sc_lut():
    mesh = plsc.VectorSubcoreMesh(
        core_axis_name="c", subcore_axis_name="s", num_cores=NC, num_subcores=NS
    )
    cp = pltpu.CompilerParams()
    if "needs_layout_passes" in pltpu.CompilerParams.__dataclass_fields__:
        cp = dataclasses.replace(cp, needs_layout_passes=False)
    return pl.kernel(
        _sc_lut_body,
        out_type=jax.ShapeDtypeStruct((NROWS, BLK), jnp.float32),
        mesh=mesh,
        scratch_types=[
            pltpu.VMEM((256,), jnp.float32),
            pltpu.VMEM((256,), jnp.float32),
        ],
        compiler_params=cp,
    )


def _sc_lut_body(x_hbm, bid_hbm, slope_hbm, icept_hbm, out_hbm, tab_s, tab_i):
    pltpu.sync_copy(slope_hbm, tab_s)
    pltpu.sync_copy(icept_hbm, tab_i)

    def body(ibuf, bidbuf, obuf):
        baddr = bidbuf[0, pl.ds(0, L)]

        @plsc.parallel_loop(0, BLK, step=L, unroll=8)
        def _(i):
            v = ibuf[0, pl.ds(i, L)]
            t = v * jnp.float32(N_BINS - 1)
            t = jnp.minimum(t, jnp.float32(18.999998))
            t = jnp.maximum(t, jnp.float32(0.0))
            addr = baddr + t.astype(jnp.int32)
            s = plsc.load_gather(tab_s, [addr])
            a = plsc.load_gather(tab_i, [addr])
            obuf[0, pl.ds(i, L)] = a + s * v

    pltpu.emit_pipeline(
        body,
        grid=(NROWS,),
        in_specs=[
            pl.BlockSpec((1, BLK), index_map=lambda i: (i, 0)),
            pl.BlockSpec((1, L), index_map=lambda i: (i, 0)),
        ],
        out_specs=[pl.BlockSpec((1, BLK), index_map=lambda i: (i, 0))],
        core_axis_name=("c", "s"),
        dimension_semantics=(pltpu.PARALLEL,),
    )(x_hbm, bid_hbm, out_hbm)


def kernel(x, ran_y):
    slope, icept = _make_tables(ran_y.astype(jnp.float32))
    bid = jnp.broadcast_to(
        ((jnp.arange(NROWS, dtype=jnp.int32) // ROWS_PER_BATCH) * 32)[:, None],
        (NROWS, L)).astype(jnp.int32)
    out = _build_sc_lut()(x.reshape(NROWS, BLK), bid,
                          slope[:, :32].reshape(-1), icept[:, :32].reshape(-1))
    return out.reshape(x.shape)


# speedup vs baseline: 1.4598x; 1.4598x over previous
import dataclasses
import functools

import jax
import jax.numpy as jnp
from jax import lax
from jax.experimental import pallas as pl
from jax.experimental.pallas import tpu as pltpu
from jax.experimental.pallas import tpu_sc as plsc

N_BINS = 20
STRENGTH = 0.7

BS = 8
TOTAL = BS * 128 * 128 * 128
NC, NS, L = 2, 16, 16
NW = NC * NS
PER_W = TOTAL // NW
BLK = 8192
NBLK = PER_W // BLK
DEPTH = 4


def _table_body(ry_ref, rysh_ref, slope_ref, icept_ref):
    col = lax.broadcasted_iota(jnp.int32, (BS, 128), 1).astype(jnp.float32)
    step = jnp.float32(1.0 / (N_BINS - 1))
    lin0 = col * step
    lin1 = (col + 1.0) * step
    w = jnp.float32(STRENGTH)
    y0 = ry_ref[...] * w + lin0 * (1.0 - w)
    y1 = rysh_ref[...] * w + lin1 * (1.0 - w)
    valid = col < float(N_BINS)
    big = jnp.float32(1e30)
    ymin = jnp.min(jnp.where(valid, y0, big), axis=1, keepdims=True)
    ymax = jnp.max(jnp.where(valid, y0, -big), axis=1, keepdims=True)
    d = ymax - ymin + jnp.float32(1e-5)
    y0n = (y0 - ymin) / d
    y1n = (y1 - ymin) / d
    slope = (y1n - y0n) / (lin1 - lin0)
    slope_ref[...] = slope
    icept_ref[...] = y0n - slope * lin0


def _make_tables(ran_y):
    ry = jnp.zeros((BS, 128), jnp.float32).at[:, :N_BINS].set(ran_y)
    rysh = jnp.zeros((BS, 128), jnp.float32).at[:, : N_BINS - 1].set(ran_y[:, 1:])
    return pl.pallas_call(
        _table_body,
        out_shape=(
            jax.ShapeDtypeStruct((BS, 128), jnp.float32),
            jax.ShapeDtypeStruct((BS, 128), jnp.float32),
        ),
    )(ry, rysh)


@functools.cache
def _build_sc_lut():
    mesh = plsc.VectorSubcoreMesh(
        core_axis_name="c", subcore_axis_name="s", num_cores=NC, num_subcores=NS
    )
    cp = pltpu.CompilerParams()
    if "needs_layout_passes" in pltpu.CompilerParams.__dataclass_fields__:
        cp = dataclasses.replace(cp, needs_layout_passes=False)
    return pl.kernel(
        _sc_lut_body,
        out_type=jax.ShapeDtypeStruct((TOTAL,), jnp.float32),
        mesh=mesh,
        scratch_types=[
            pltpu.VMEM((128,), jnp.float32),
            pltpu.VMEM((128,), jnp.float32),
            pltpu.VMEM((DEPTH, BLK), jnp.float32),
            pltpu.VMEM((DEPTH, BLK), jnp.float32),
        ] + [pltpu.SemaphoreType.DMA] * (2 * DEPTH),
        compiler_params=cp,
    )


def _sc_lut_body(x_hbm, slope_hbm, icept_hbm, out_hbm,
                 tab_s, tab_i, ibuf, obuf, *sems):
    wid = lax.axis_index("s") * NC + lax.axis_index("c")
    batch = wid // (NW // BS)
    base = wid * PER_W
    pltpu.sync_copy(slope_hbm.at[batch], tab_s)
    pltpu.sync_copy(icept_hbm.at[batch], tab_i)
    isems = sems[:DEPTH]
    osems = sems[DEPTH:]

    def in_copy(jj, slot):
        return pltpu.make_async_copy(
            x_hbm.at[pl.ds(base + jj * BLK, BLK)], ibuf.at[slot], isems[slot])

    def out_copy(jj, slot):
        return pltpu.make_async_copy(
            obuf.at[slot], out_hbm.at[pl.ds(base + jj * BLK, BLK)], osems[slot])

    def compute(slot):
        @plsc.parallel_loop(0, BLK, step=L, unroll=8)
        def _(i):
            v = ibuf[slot, pl.ds(i, L)]
            t = v * jnp.float32(N_BINS - 1)
            t = jnp.minimum(t, jnp.float32(18.999998))
            t = jnp.maximum(t, jnp.float32(0.0))
            idx = t.astype(jnp.int32)
            s = plsc.load_gather(tab_s, [idx])
            a = plsc.load_gather(tab_i, [idx])
            obuf[slot, pl.ds(i, L)] = a + s * v

    for slot in range(DEPTH):
        in_copy(slot, slot).start()

    @pl.loop(0, NBLK, step=DEPTH)
    def _(j):
        for slot in range(DEPTH):
            jj = j + slot
            in_copy(jj, slot).wait()

            @pl.when(jj >= DEPTH)
            def _():
                out_copy(jj - DEPTH, slot).wait()

            compute(slot)
            out_copy(jj, slot).start()

            @pl.when(jj + DEPTH < NBLK)
            def _():
                in_copy(jj + DEPTH, slot).start()

    for slot in range(DEPTH):
        out_copy(NBLK - DEPTH + slot, slot).wait()


def kernel(x, ran_y):
    slope, icept = _make_tables(ran_y.astype(jnp.float32))
    out_flat = _build_sc_lut()(x.reshape(TOTAL), slope, icept)
    return out_flat.reshape(x.shape)


# speedup vs baseline: 1.4666x; 1.0046x over previous
import dataclasses
import functools

import jax
import jax.numpy as jnp
from jax import lax
from jax.experimental import pallas as pl
from jax.experimental.pallas import tpu as pltpu
from jax.experimental.pallas import tpu_sc as plsc

N_BINS = 20
STRENGTH = 0.7

BS = 8
TOTAL = BS * 128 * 128 * 128
NC, NS, L = 2, 16, 16
NW = NC * NS
PER_W = TOTAL // NW
BLK = 16384
NBLK = PER_W // BLK


def _table_body(ry_ref, rysh_ref, slope_ref, icept_ref):
    col = lax.broadcasted_iota(jnp.int32, (BS, 128), 1).astype(jnp.float32)
    step = jnp.float32(1.0 / (N_BINS - 1))
    lin0 = col * step
    lin1 = (col + 1.0) * step
    w = jnp.float32(STRENGTH)
    y0 = ry_ref[...] * w + lin0 * (1.0 - w)
    y1 = rysh_ref[...] * w + lin1 * (1.0 - w)
    valid = col < float(N_BINS)
    big = jnp.float32(1e30)
    ymin = jnp.min(jnp.where(valid, y0, big), axis=1, keepdims=True)
    ymax = jnp.max(jnp.where(valid, y0, -big), axis=1, keepdims=True)
    d = ymax - ymin + jnp.float32(1e-5)
    y0n = (y0 - ymin) / d
    y1n = (y1 - ymin) / d
    slope = (y1n - y0n) / (lin1 - lin0)
    slope_ref[...] = slope
    icept_ref[...] = y0n - slope * lin0


def _make_tables(ran_y):
    ry = jnp.zeros((BS, 128), jnp.float32).at[:, :N_BINS].set(ran_y)
    rysh = jnp.zeros((BS, 128), jnp.float32).at[:, : N_BINS - 1].set(ran_y[:, 1:])
    return pl.pallas_call(
        _table_body,
        out_shape=(
            jax.ShapeDtypeStruct((BS, 128), jnp.float32),
            jax.ShapeDtypeStruct((BS, 128), jnp.float32),
        ),
    )(ry, rysh)


@functools.cache
def _build_sc_lut():
    mesh = plsc.VectorSubcoreMesh(
        core_axis_name="c", subcore_axis_name="s", num_cores=NC, num_subcores=NS
    )
    cp = pltpu.CompilerParams()
    if "needs_layout_passes" in pltpu.CompilerParams.__dataclass_fields__:
        cp = dataclasses.replace(cp, needs_layout_passes=False)
    return pl.kernel(
        _sc_lut_body,
        out_type=jax.ShapeDtypeStruct((TOTAL,), jnp.float32),
        mesh=mesh,
        scratch_types=[
            pltpu.VMEM((128,), jnp.float32),
            pltpu.VMEM((128,), jnp.float32),
            pltpu.VMEM((2, BLK), jnp.float32),
            pltpu.VMEM((2, BLK), jnp.float32),
            pltpu.SemaphoreType.DMA,
            pltpu.SemaphoreType.DMA,
            pltpu.SemaphoreType.DMA,
            pltpu.SemaphoreType.DMA,
        ],
        compiler_params=cp,
    )


def _sc_lut_body(x_hbm, slope_hbm, icept_hbm, out_hbm,
                 tab_s, tab_i, ibuf, obuf, si0, si1, so0, so1):
    wid = lax.axis_index("s") * NC + lax.axis_index("c")
    batch = wid // (NW // BS)
    base = wid * PER_W
    pltpu.sync_copy(slope_hbm.at[batch], tab_s)
    pltpu.sync_copy(icept_hbm.at[batch], tab_i)
    isems = (si0, si1)
    osems = (so0, so1)

    def in_copy(jj, slot):
        return pltpu.make_async_copy(
            x_hbm.at[pl.ds(base + jj * BLK, BLK)], ibuf.at[slot], isems[slot])

    def out_copy(jj, slot):
        return pltpu.make_async_copy(
            obuf.at[slot], out_hbm.at[pl.ds(base + jj * BLK, BLK)], osems[slot])

    def compute(slot):
        @plsc.parallel_loop(0, BLK, step=L, unroll=8)
        def _(i):
            v = ibuf[slot, pl.ds(i, L)]
            t = v * jnp.float32(N_BINS - 1)
            t = jnp.minimum(t, jnp.float32(18.999998))
            t = jnp.maximum(t, jnp.float32(0.0))
            idx = t.astype(jnp.int32)
            s = plsc.load_gather(tab_s, [idx])
            a = plsc.load_gather(tab_i, [idx])
            obuf[slot, pl.ds(i, L)] = a + s * v

    in_copy(0, 0).start()
    in_copy(1, 1).start()

    @pl.loop(0, NBLK, step=2)
    def _(j):
        for slot in range(2):
            jj = j + slot
            in_copy(jj, slot).wait()

            @pl.when(jj >= 2)
            def _():
                out_copy(jj - 2, slot).wait()

            compute(slot)
            out_copy(jj, slot).start()

            @pl.when(jj + 2 < NBLK)
            def _():
                in_copy(jj + 2, slot).start()

    out_copy(NBLK - 2, 0).wait()
    out_copy(NBLK - 1, 1).wait()


def kernel(x, ran_y):
    slope, icept = _make_tables(ran_y.astype(jnp.float32))
    out_flat = _build_sc_lut()(x.reshape(TOTAL), slope, icept)
    return out_flat.reshape(x.shape)
